# baseline (speedup 1.0000x reference)
import functools

import jax
import jax.numpy as jnp
from jax import lax
from jax.experimental import pallas as pl
from jax.experimental.pallas import tpu as pltpu
from jax.experimental.pallas import tpu_sc as plsc

B = 4
J = 21
V = 32
P = B * J
PC = P // 2
BC = B // 2
N = V * V * V
G = 2 * V
L = 16
NS = 16
XB = 8
NCH = G // XB
CW = XB * G * G

_MESH = plsc.VectorSubcoreMesh(core_axis_name="c", subcore_axis_name="s")
_PARAMS = pltpu.CompilerParams(needs_layout_passes=False)


@functools.partial(
    pl.kernel,
    out_type=(
        jax.ShapeDtypeStruct((P, G * G * G), jnp.float32),
        jax.ShapeDtypeStruct((B * L,), jnp.float32),
        jax.ShapeDtypeStruct((B * L,), jnp.float32),
    ),
    mesh=_MESH,
    compiler_params=_PARAMS,
    scratch_types=[
        pltpu.VMEM((N,), jnp.float32),
        pltpu.VMEM((CW,), jnp.float32),
        pltpu.VMEM((CW,), jnp.float32),
        pltpu.VMEM((L,), jnp.float32),
        pltpu.VMEM((L,), jnp.float32),
        pltpu.VMEM((L,), jnp.int32),
        pltpu.VMEM_SHARED((PC * L,), jnp.float32),
        pltpu.VMEM((PC * L,), jnp.float32),
        pltpu.VMEM((B * L,), jnp.float32),
        pltpu.VMEM((B * L,), jnp.float32),
        pltpu.VMEM((G,), jnp.float32),
        pltpu.VMEM((BC * 3 * G,), jnp.int32),
        pltpu.VMEM((BC * L,), jnp.float32),
        pltpu.VMEM((BC * L,), jnp.float32),
        pltpu.VMEM((G * G,), jnp.int32),
        pltpu.SemaphoreType.DMA,
        pltpu.SemaphoreType.DMA,
    ],
)
def _crop_kernel(hm_hbm, vmin_hbm, vmax_hbm, t_hbm,
                 out_hbm, maxb_hbm, minb_hbm,
                 hbuf, obuf0, obuf1, pbuf, rbuf, ribuf,
                 pshared, posv, vminv, vmaxv, tv, idxv, mbv, nbv,
                 rowbase, sem0, sem1):
    c = lax.axis_index("c")
    s = lax.axis_index("s")
    lane = lax.iota(jnp.int32, L)
    rots = [jnp.bitwise_and(lane + r, L - 1) for r in (1, 2, 4, 8)]
    U = 8

    def _allmax(v):
        for r in rots:
            rbuf[...] = v
            v = jnp.maximum(v, plsc.load_gather(rbuf, [r]))
        return v

    def _allsum_i(v):
        for r in rots:
            ribuf[...] = v
            v = v + plsc.load_gather(ribuf, [r])
        return v

    for k in range(3):
        pl_ = s + NS * k

        @pl.when(pl_ < PC)
        def _():
            pg = c * PC + pl_
            pltpu.sync_copy(hm_hbm.at[pg], hbuf)

            @plsc.parallel_loop(0, N // (U * L), carry=hbuf[pl.ds(0, L)],
                                unroll=2)
            def m0(i, m):
                for u in range(U):
                    m = jnp.maximum(m, hbuf[pl.ds(i * U * L + u * L, L)])
                return m

            mx = _allmax(m0)

            z = jnp.zeros((L,), jnp.int32)

            @plsc.parallel_loop(0, N // (U * L), carry=(z, z, z, z), unroll=2)
            def sums(i, carry):
                si, sj, sk, cnt = carry
                for u in range(U):
                    v = hbuf[pl.ds(i * U * L + u * L, L)]
                    nn = i * (U * L) + u * L + lane
                    e = v == mx
                    si = si + jnp.where(e, lax.shift_right_logical(nn, 10), 0)
                    sj = sj + jnp.where(e, jnp.bitwise_and(lax.shift_right_logical(nn, 5), 31), 0)
                    sk = sk + jnp.where(e, jnp.bitwise_and(nn, 31), 0)
                    cnt = cnt + jnp.where(e, 1, 0)
                return si, sj, sk, cnt

            si, sj, sk, cnt = sums
            cf = _allsum_i(cnt).astype(jnp.float32)
            pi = _allsum_i(si).astype(jnp.float32) / cf
            pj = _allsum_i(sj).astype(jnp.float32) / cf
            pk = _allsum_i(sk).astype(jnp.float32) / cf
            res = (jnp.where(lane == 0, pi, 0.0)
                   + jnp.where(lane == 1, pj, 0.0)
                   + jnp.where(lane == 2, pk, 0.0))
            pbuf[...] = res
            pltpu.sync_copy(pbuf, pshared.at[pl.ds(pl_ * L, L)])

    plsc.subcore_barrier()

    pltpu.sync_copy(pshared, posv)
    pltpu.sync_copy(vmin_hbm, vminv)
    pltpu.sync_copy(vmax_hbm, vmaxv)
    pltpu.sync_copy(t_hbm, tv)
    for bl in range(BC):
        mxv = posv[pl.ds(bl * J * L, L)]
        mnv = mxv
        for j in range(1, J):
            r = posv[pl.ds((bl * J + j) * L, L)]
            mxv = jnp.maximum(mxv, r)
            mnv = jnp.minimum(mnv, r)
        mxb = jnp.minimum(jnp.maximum(mxv + 3.0, 0.0), 31.0)
        mnb = jnp.minimum(jnp.maximum(mnv - 3.0, 0.0), 31.0)
        bg = 2 * c + bl
        vmin_row = vminv[pl.ds(bg * L, L)]
        dv = vmaxv[pl.ds(bg * L, L)] - vmin_row
        maxbv = vmin_row + mxb / 31.0 * dv
        minbv = vmin_row + mnb / 31.0 * dv
        itv_v = dv / 31.0
        mbv[pl.ds(bl * L, L)] = maxbv
        nbv[pl.ds(bl * L, L)] = minbv
        for ax in range(3):
            mxb_s = maxbv[ax]
            mnb_s = minbv[ax]
            vmin_s = vmin_row[ax]
            itv_s = itv_v[ax]
            for q in range(G // L):
                tt = tv[pl.ds(q * L, L)]
                gx = mnb_s + tt * (mxb_s - mnb_s)
                vox = (gx - vmin_s) / itv_s
                ni = vox.astype(jnp.int32)
                frac = vox - ni.astype(jnp.float32)
                up = (frac > 0.5) | ((frac == 0.5) & ((ni & 1) == 1))
                r = jnp.minimum(jnp.maximum(ni + jnp.where(up, 1, 0), 0), 31)
                idxv[pl.ds((bl * 3 + ax) * G + q * L, L)] = r

    @pl.when(s == 0)
    def _():
        pltpu.sync_copy(mbv, maxb_hbm.at[pl.ds(c * BC * L, BC * L)])
        pltpu.sync_copy(nbv, minb_hbm.at[pl.ds(c * BC * L, BC * L)])

    bufs = (obuf0, obuf1)
    sems = (sem0, sem1)

    def pair_body(k, _):
        pl_ = s + NS * k

        @pl.when(pl_ < PC)
        def _():
            pg = c * PC + pl_
            bl = pl_ // J
            ib = bl * 3 * G
            pltpu.sync_copy(hm_hbm.at[pg], hbuf)
            izq = [idxv[pl.ds(ib + 2 * G + q * L, L)] for q in range(G // L)]
            yoq = [idxv[pl.ds(ib + G + q * L, L)] * V for q in range(G // L)]
            for xq in range(G // L):
                xv = idxv[pl.ds(ib + xq * L, L)] * (V * V)
                for l in range(L):
                    xs = xv[l]
                    for q in range(G // L):
                        rowbase[pl.ds((xq * L + l) * G + q * L, L)] = xs + yoq[q]

            def chunk_body(h, _):
                for sb in range(2):
                    xb = 2 * h + sb

                    @pl.when(h > 0)
                    def _():
                        pltpu.make_async_copy(
                            bufs[sb], out_hbm.at[pg, pl.ds((xb - 2) * CW, CW)],
                            sems[sb]).wait()

                    buf = bufs[sb]

                    @plsc.parallel_loop(0, XB * G // L, unroll=2)
                    def _(g):
                        bases = rowbase[pl.ds(xb * XB * G + g * L, L)]
                        for l in range(L):
                            base_s = bases[l]
                            for q in range(G // L):
                                vals = plsc.load_gather(hbuf, [base_s + izq[q]])
                                buf[pl.ds((g * L + l) * G + q * L, L)] = vals
                    pltpu.async_copy(buf, out_hbm.at[pg, pl.ds(xb * CW, CW)],
                                     sems[sb])
                return 0

            lax.fori_loop(0, NCH // 2, chunk_body, 0)
            for sb in range(2):
                pltpu.make_async_copy(
                    bufs[sb], out_hbm.at[pg, pl.ds((NCH - 2 + sb) * CW, CW)],
                    sems[sb]).wait()
        return 0

    lax.fori_loop(0, 3, pair_body, 0)


def kernel(heatmap, vmin_s1, vmax, vmin):
    del vmin_s1
    hm2 = heatmap.reshape(P, N)
    vminp = jnp.pad(vmin[:, 0, :], ((0, 0), (0, L - 3))).reshape(B * L)
    vmaxp = jnp.pad(vmax[:, 0, :], ((0, 0), (0, L - 3))).reshape(B * L)
    t = jnp.linspace(0.0, 1.0, G)
    out, maxb, minb = _crop_kernel(hm2, vminp, vmaxp, t)
    interp = out.reshape(B, J, G, G, G)
    max_b = maxb.reshape(B, L)[:, :3].reshape(B, 1, 3)
    min_b = minb.reshape(B, L)[:, :3].reshape(B, 1, 3)
    return interp, max_b, min_b


# speedup vs baseline: 1.0039x; 1.0039x over previous
import functools

import jax
import jax.numpy as jnp
from jax import lax
from jax.experimental import pallas as pl
from jax.experimental.pallas import tpu as pltpu
from jax.experimental.pallas import tpu_sc as plsc

B = 4
J = 21
V = 32
P = B * J
PC = P // 2
BC = B // 2
N = V * V * V
G = 2 * V
L = 16
NS = 16
XB = 8
NCH = G // XB
CW = XB * G * G

_MESH = plsc.VectorSubcoreMesh(core_axis_name="c", subcore_axis_name="s")
_PARAMS = pltpu.CompilerParams(needs_layout_passes=False)


@functools.partial(
    pl.kernel,
    out_type=(
        jax.ShapeDtypeStruct((P, G * G * G), jnp.float32),
        jax.ShapeDtypeStruct((B * L,), jnp.float32),
        jax.ShapeDtypeStruct((B * L,), jnp.float32),
    ),
    mesh=_MESH,
    compiler_params=_PARAMS,
    scratch_types=[
        pltpu.VMEM((N,), jnp.float32),
        pltpu.VMEM((CW,), jnp.float32),
        pltpu.VMEM((CW,), jnp.float32),
        pltpu.VMEM((L,), jnp.float32),
        pltpu.VMEM((L,), jnp.float32),
        pltpu.VMEM((L,), jnp.int32),
        pltpu.VMEM_SHARED((PC * L,), jnp.float32),
        pltpu.VMEM((PC * L,), jnp.float32),
        pltpu.VMEM((B * L,), jnp.float32),
        pltpu.VMEM((B * L,), jnp.float32),
        pltpu.VMEM((G,), jnp.float32),
        pltpu.VMEM((BC * 3 * G,), jnp.int32),
        pltpu.VMEM((BC * L,), jnp.float32),
        pltpu.VMEM((BC * L,), jnp.float32),
        pltpu.VMEM((G * G,), jnp.int32),
        pltpu.SemaphoreType.DMA,
        pltpu.SemaphoreType.DMA,
    ],
)
def _crop_kernel(hm_hbm, vmin_hbm, vmax_hbm, t_hbm,
                 out_hbm, maxb_hbm, minb_hbm,
                 hbuf, obuf0, obuf1, pbuf, rbuf, ribuf,
                 pshared, posv, vminv, vmaxv, tv, idxv, mbv, nbv,
                 rowbase, sem0, sem1):
    c = lax.axis_index("c")
    s = lax.axis_index("s")
    lane = lax.iota(jnp.int32, L)
    rots = [jnp.bitwise_and(lane + r, L - 1) for r in (1, 2, 4, 8)]
    U = 4

    def _allmax(v):
        for r in rots:
            rbuf[...] = v
            v = jnp.maximum(v, plsc.load_gather(rbuf, [r]))
        return v

    def _allsum_i(v):
        for r in rots:
            ribuf[...] = v
            v = v + plsc.load_gather(ribuf, [r])
        return v

    for k in range(3):
        pl_ = s + NS * k

        @pl.when(pl_ < PC)
        def _():
            pg = c * PC + pl_
            pltpu.sync_copy(hm_hbm.at[pg], hbuf)

            @plsc.parallel_loop(0, N // (U * L), carry=hbuf[pl.ds(0, L)],
                                unroll=2)
            def m0(i, m):
                for u in range(U):
                    m = jnp.maximum(m, hbuf[pl.ds(i * U * L + u * L, L)])
                return m

            mx = _allmax(m0)

            z = jnp.zeros((L,), jnp.int32)

            @plsc.parallel_loop(0, N // (U * L), carry=(z, z, z, z), unroll=2)
            def sums(i, carry):
                si, sj, sk, cnt = carry
                for u in range(U):
                    v = hbuf[pl.ds(i * U * L + u * L, L)]
                    nn = i * (U * L) + u * L + lane
                    e = v == mx
                    si = si + jnp.where(e, lax.shift_right_logical(nn, 10), 0)
                    sj = sj + jnp.where(e, jnp.bitwise_and(lax.shift_right_logical(nn, 5), 31), 0)
                    sk = sk + jnp.where(e, jnp.bitwise_and(nn, 31), 0)
                    cnt = cnt + jnp.where(e, 1, 0)
                return si, sj, sk, cnt

            si, sj, sk, cnt = sums
            cf = _allsum_i(cnt).astype(jnp.float32)
            pi = _allsum_i(si).astype(jnp.float32) / cf
            pj = _allsum_i(sj).astype(jnp.float32) / cf
            pk = _allsum_i(sk).astype(jnp.float32) / cf
            res = (jnp.where(lane == 0, pi, 0.0)
                   + jnp.where(lane == 1, pj, 0.0)
                   + jnp.where(lane == 2, pk, 0.0))
            pbuf[...] = res
            pltpu.sync_copy(pbuf, pshared.at[pl.ds(pl_ * L, L)])

    plsc.subcore_barrier()

    pltpu.sync_copy(pshared, posv)
    pltpu.sync_copy(vmin_hbm, vminv)
    pltpu.sync_copy(vmax_hbm, vmaxv)
    pltpu.sync_copy(t_hbm, tv)
    for bl in range(BC):
        mxv = posv[pl.ds(bl * J * L, L)]
        mnv = mxv
        for j in range(1, J):
            r = posv[pl.ds((bl * J + j) * L, L)]
            mxv = jnp.maximum(mxv, r)
            mnv = jnp.minimum(mnv, r)
        mxb = jnp.minimum(jnp.maximum(mxv + 3.0, 0.0), 31.0)
        mnb = jnp.minimum(jnp.maximum(mnv - 3.0, 0.0), 31.0)
        bg = 2 * c + bl
        vmin_row = vminv[pl.ds(bg * L, L)]
        dv = vmaxv[pl.ds(bg * L, L)] - vmin_row
        maxbv = vmin_row + mxb / 31.0 * dv
        minbv = vmin_row + mnb / 31.0 * dv
        itv_v = dv / 31.0
        mbv[pl.ds(bl * L, L)] = maxbv
        nbv[pl.ds(bl * L, L)] = minbv
        for ax in range(3):
            mxb_s = maxbv[ax]
            mnb_s = minbv[ax]
            vmin_s = vmin_row[ax]
            itv_s = itv_v[ax]
            for q in range(G // L):
                tt = tv[pl.ds(q * L, L)]
                gx = mnb_s + tt * (mxb_s - mnb_s)
                vox = (gx - vmin_s) / itv_s
                ni = vox.astype(jnp.int32)
                frac = vox - ni.astype(jnp.float32)
                up = (frac > 0.5) | ((frac == 0.5) & ((ni & 1) == 1))
                r = jnp.minimum(jnp.maximum(ni + jnp.where(up, 1, 0), 0), 31)
                idxv[pl.ds((bl * 3 + ax) * G + q * L, L)] = r

    @pl.when(s == 0)
    def _():
        pltpu.sync_copy(mbv, maxb_hbm.at[pl.ds(c * BC * L, BC * L)])
        pltpu.sync_copy(nbv, minb_hbm.at[pl.ds(c * BC * L, BC * L)])

    bufs = (obuf0, obuf1)
    sems = (sem0, sem1)

    def pair_body(k, _):
        pl_ = s + NS * k

        @pl.when(pl_ < PC)
        def _():
            pg = c * PC + pl_
            bl = pl_ // J
            ib = bl * 3 * G
            pltpu.sync_copy(hm_hbm.at[pg], hbuf)
            izq = [idxv[pl.ds(ib + 2 * G + q * L, L)] for q in range(G // L)]
            yoq = [idxv[pl.ds(ib + G + q * L, L)] * V for q in range(G // L)]
            for xq in range(G // L):
                xv = idxv[pl.ds(ib + xq * L, L)] * (V * V)
                for l in range(L):
                    xs = xv[l]
                    for q in range(G // L):
                        rowbase[pl.ds((xq * L + l) * G + q * L, L)] = xs + yoq[q]

            def chunk_body(h, _):
                for sb in range(2):
                    xb = 2 * h + sb

                    @pl.when(h > 0)
                    def _():
                        pltpu.make_async_copy(
                            bufs[sb], out_hbm.at[pg, pl.ds((xb - 2) * CW, CW)],
                            sems[sb]).wait()

                    buf = bufs[sb]

                    @plsc.parallel_loop(0, XB * G // L, unroll=2)
                    def _(g):
                        bases = rowbase[pl.ds(xb * XB * G + g * L, L)]
                        for l in range(L):
                            base_s = bases[l]
                            for q in range(G // L):
                                vals = plsc.load_gather(hbuf, [base_s + izq[q]])
                                buf[pl.ds((g * L + l) * G + q * L, L)] = vals
                    pltpu.async_copy(buf, out_hbm.at[pg, pl.ds(xb * CW, CW)],
                                     sems[sb])
                return 0

            lax.fori_loop(0, NCH // 2, chunk_body, 0)
            for sb in range(2):
                pltpu.make_async_copy(
                    bufs[sb], out_hbm.at[pg, pl.ds((NCH - 2 + sb) * CW, CW)],
                    sems[sb]).wait()
        return 0

    lax.fori_loop(0, 3, pair_body, 0)


def kernel(heatmap, vmin_s1, vmax, vmin):
    del vmin_s1
    hm2 = heatmap.reshape(P, N)
    vminp = jnp.pad(vmin[:, 0, :], ((0, 0), (0, L - 3))).reshape(B * L)
    vmaxp = jnp.pad(vmax[:, 0, :], ((0, 0), (0, L - 3))).reshape(B * L)
    t = jnp.linspace(0.0, 1.0, G)
    out, maxb, minb = _crop_kernel(hm2, vminp, vmaxp, t)
    interp = out.reshape(B, J, G, G, G)
    max_b = maxb.reshape(B, L)[:, :3].reshape(B, 1, 3)
    min_b = minb.reshape(B, L)[:, :3].reshape(B, 1, 3)
    return interp, max_b, min_b
